# Initial kernel scaffold; baseline (speedup 1.0000x reference)
import jax
import jax.numpy as jnp
from jax.experimental import pallas as pl


def kernel(x, table):
    raise NotImplementedError("write your pallas kernel here")


# speedup vs baseline: 1.7981x; 1.7981x over previous
import functools
import jax
import jax.numpy as jnp
from jax import lax
from jax.experimental import pallas as pl
from jax.experimental.pallas import tpu as pltpu
from jax.experimental.pallas import tpu_sc as plsc

BATCH = 16384
HIST = 50
EMBED_DIM = 64
TOTAL = BATCH * HIST

NUM_WORKERS = 32
ROWS_PER_WORKER = TOTAL // NUM_WORKERS
CHUNK = 512
KSUB = CHUNK // 128
NCHUNKS = ROWS_PER_WORKER // CHUNK


def _sc_gather(x2d, table):
    mesh = plsc.VectorSubcoreMesh(core_axis_name="c", subcore_axis_name="s")

    @functools.partial(
        pl.kernel,
        mesh=mesh,
        out_type=jax.ShapeDtypeStruct((TOTAL, EMBED_DIM), jnp.float32),
        scratch_types=[
            pltpu.VMEM((KSUB, 128), jnp.int32),
            pltpu.VMEM((CHUNK, EMBED_DIM), jnp.float32),
            pltpu.SemaphoreType.DMA,
        ],
        compiler_params=pltpu.CompilerParams(use_tc_tiling_on_sc=False),
    )
    def k(x_hbm, table_hbm, out_hbm, idx_v, rows_v, sem):
        wid = lax.axis_index("s") * 2 + lax.axis_index("c")
        row0 = wid * (ROWS_PER_WORKER // 128)

        def chunk_body(ch, carry):
            idx_row = row0 + ch * KSUB
            pltpu.sync_copy(x_hbm.at[pl.ds(idx_row, KSUB)], idx_v)
            copies = []
            for j in range(KSUB):
                copies.append(pltpu.async_copy(
                    table_hbm.at[idx_v.at[j]],
                    rows_v.at[pl.ds(j * 128, 128)],
                    sem))
            for c in copies:
                c.wait()
            pltpu.sync_copy(rows_v,
                            out_hbm.at[pl.ds(idx_row * 128, CHUNK)])
            return carry

        lax.fori_loop(0, NCHUNKS, chunk_body, 0)

    return k(x2d, table)


def kernel(x, table):
    x2d = jnp.reshape(x.astype(jnp.int32), (TOTAL // 128, 128))
    out = _sc_gather(x2d, table)
    return jnp.reshape(out, (BATCH, HIST, EMBED_DIM))


# speedup vs baseline: 1.8776x; 1.0442x over previous
import functools
import jax
import jax.numpy as jnp
from jax import lax
from jax.experimental import pallas as pl
from jax.experimental.pallas import tpu as pltpu
from jax.experimental.pallas import tpu_sc as plsc

BATCH = 16384
HIST = 50
EMBED_DIM = 64
TOTAL = BATCH * HIST

NUM_WORKERS = 32
ROWS_PER_WORKER = TOTAL // NUM_WORKERS
CHUNK = 512
KSUB = CHUNK // 128
NCHUNKS = ROWS_PER_WORKER // CHUNK


def _sc_gather(x2d, table):
    mesh = plsc.VectorSubcoreMesh(core_axis_name="c", subcore_axis_name="s")

    @functools.partial(
        pl.kernel,
        mesh=mesh,
        out_type=jax.ShapeDtypeStruct((TOTAL, EMBED_DIM), jnp.float32),
        scratch_types=[
            pltpu.VMEM((2, KSUB, 128), jnp.int32),
            pltpu.VMEM((2, CHUNK, EMBED_DIM), jnp.float32),
            pltpu.SemaphoreType.DMA((2,)),
            pltpu.SemaphoreType.DMA((2,)),
        ],
        compiler_params=pltpu.CompilerParams(use_tc_tiling_on_sc=False),
    )
    def k(x_hbm, table_hbm, out_hbm, idx_v, rows_v, gsem, wsem):
        wid = lax.axis_index("s") * 2 + lax.axis_index("c")
        row0 = wid * (ROWS_PER_WORKER // 128)

        def fire_gathers(ch, b):
            for j in range(KSUB):
                pltpu.async_copy(
                    table_hbm.at[idx_v.at[b, j]],
                    rows_v.at[b, pl.ds(j * 128, 128)],
                    gsem.at[b])

        def wait_gathers(ch, b):
            for j in range(KSUB):
                pltpu.make_async_copy(
                    table_hbm.at[idx_v.at[b, j]],
                    rows_v.at[b, pl.ds(j * 128, 128)],
                    gsem.at[b]).wait()

        def start_write(ch, b):
            pltpu.async_copy(
                rows_v.at[b],
                out_hbm.at[pl.ds((row0 + ch * KSUB) * 128, CHUNK)],
                wsem.at[b])

        def wait_write(ch, b):
            pltpu.make_async_copy(
                rows_v.at[b],
                out_hbm.at[pl.ds((row0 + ch * KSUB) * 128, CHUNK)],
                wsem.at[b]).wait()

        def load_idx(ch, b):
            pltpu.sync_copy(x_hbm.at[pl.ds(row0 + ch * KSUB, KSUB)],
                            idx_v.at[b])

        load_idx(0, 0)
        fire_gathers(0, 0)
        load_idx(1, 1)
        fire_gathers(1, 1)

        def step(t, carry):
            for b in range(2):
                ch = 2 * t + b
                wait_gathers(ch, b)
                start_write(ch, b)
                load_idx(ch + 2, b)
                wait_write(ch, b)
                fire_gathers(ch + 2, b)
            return carry

        lax.fori_loop(0, NCHUNKS // 2 - 1, step, 0)

        for b in range(2):
            ch = NCHUNKS - 2 + b
            wait_gathers(ch, b)
            start_write(ch, b)
        for b in range(2):
            wait_write(NCHUNKS - 2 + b, b)

    return k(x2d, table)


def kernel(x, table):
    x2d = jnp.reshape(x.astype(jnp.int32), (TOTAL // 128, 128))
    out = _sc_gather(x2d, table)
    return jnp.reshape(out, (BATCH, HIST, EMBED_DIM))
